# baseline (speedup 1.0000x reference)
import jax
import jax.numpy as jnp
from jax import lax
from jax.experimental import pallas as pl
from jax.experimental.pallas import tpu as pltpu
from jax.experimental.pallas import tpu_sc as plsc

NC = 2
NS = 16
L = 16
NW = NC * NS

N = 32768
ROWS = 1024
RPW = ROWS // NW
NV = N // L

G = 16
NGRP = NV // G
GE = G * L

C = 4096

BISECT = 14
REFINE = 3

_NEG = -3.0e38


def _tau_from(buf, nv, rowmax):
    def relu_sum(t):
        def body(j, acc):
            v = buf[pl.ds(j * L, L)]
            return acc + jnp.maximum(v - t, 0.0)
        acc = lax.fori_loop(0, nv, body, jnp.zeros((L,), jnp.float32))
        return jnp.sum(acc)

    lo = rowmax - 1.0
    hi = rowmax

    def bis(_, lohi):
        lo, hi = lohi
        mid = 0.5 * (lo + hi)
        big = relu_sum(mid) >= 1.0
        return (jnp.where(big, mid, lo), jnp.where(big, hi, mid))

    lo, hi = lax.fori_loop(0, BISECT, bis, (lo, hi))

    def refine(_, t16):
        def body(j, carry):
            s16, k16 = carry
            v = buf[pl.ds(j * L, L)]
            m = v > t16
            return (s16 + jnp.where(m, v, 0.0),
                    k16 + jnp.where(m, 1.0, 0.0))
        s16, k16 = lax.fori_loop(
            0, nv, body,
            (jnp.zeros((L,), jnp.float32), jnp.zeros((L,), jnp.float32)))
        num = jnp.broadcast_to(jnp.sum(s16) - 1.0, (L,))
        den = jnp.broadcast_to(jnp.sum(k16), (L,))
        return jnp.maximum(t16, num / den)

    return lax.fori_loop(0, REFINE, refine, jnp.broadcast_to(lo, (L,)))


def _phase1(xb, gmax, lcid):
    neg = jnp.full((L,), _NEG, jnp.float32)

    @plsc.parallel_loop(0, NGRP, unroll=4, carry=neg)
    def grp_max(g, acc):
        b = g * GE
        v = [xb[pl.ds(b + k * L, L)] for k in range(G)]
        m = [jnp.maximum(v[2 * i], v[2 * i + 1]) for i in range(8)]
        m = [jnp.maximum(m[2 * i], m[2 * i + 1]) for i in range(4)]
        m = [jnp.maximum(m[2 * i], m[2 * i + 1]) for i in range(2)]
        cm = jnp.maximum(m[0], m[1])
        gmax[pl.ds(g * L, L)] = cm
        return jnp.maximum(acc, cm)

    rowmax = jnp.max(grp_max)
    thr16 = jnp.broadcast_to(rowmax - 1.0, (L,))

    def live_cols(gg, off):
        for k4 in range(8):
            g = gg * 8 + k4
            m = gmax[pl.ds(g * L, L)] > thr16
            ids = lax.iota(jnp.int32, L) + g * L
            plsc.store_compressed(lcid.at[pl.ds(off, L)], ids, mask=m)
            off = off + plsc.all_reduce_population_count(m)[0]
        return off

    nlive = lax.fori_loop(0, NGRP // 8, live_cols, 0)
    lcid[pl.ds(nlive, L)] = jnp.zeros((L,), jnp.int32)
    return rowmax, nlive


def _phase2(xb, cbuf, lcid, nlive, rowmax):
    thr16 = jnp.broadcast_to(rowmax - 1.0, (L,))
    nl16 = (nlive + (L - 1)) // L

    def gather_cols(i, carry):
        off, true_cnt = carry
        w = lcid[pl.ds(i * L, L)]
        lane_ok = (lax.iota(jnp.int32, L) + i * L) < nlive
        base = jnp.right_shift(w, 4) * GE + jnp.bitwise_and(w, 15)
        for k in range(G):
            val = plsc.load_gather(xb, [base + k * L])
            mk = jnp.logical_and(val > thr16, lane_ok)
            fits = jnp.broadcast_to(off <= C - L, (L,))
            ms = jnp.logical_and(mk, fits)
            plsc.store_compressed(cbuf.at[pl.ds(off, L)], val, mask=ms)
            off = off + plsc.all_reduce_population_count(ms)[0]
            true_cnt = true_cnt + plsc.all_reduce_population_count(mk)[0]
        return off, true_cnt

    cnt, true_cnt = lax.fori_loop(0, nl16, gather_cols, (0, 0))
    cbuf[pl.ds(cnt, L)] = jnp.full((L,), _NEG, jnp.float32)

    nvc = (cnt + (L - 1)) // L
    return lax.cond(
        true_cnt == cnt,
        lambda: _tau_from(cbuf, nvc, rowmax),
        lambda: _tau_from(xb, NV, rowmax))


def _emit_patches(xb, lcid, nlive, tau16, out_row, psem):
    def patch(i, carry):
        prevg, npat = carry
        g = jnp.right_shift(lcid[pl.ds(i, L)][0], 4)

        def emit(npat):
            b = g * GE
            for k in range(G):
                v = xb[pl.ds(b + k * L, L)]
                xb[pl.ds(b + k * L, L)] = jnp.maximum(v - tau16, 0.0)
            pltpu.async_copy(xb.at[pl.ds(b, GE)],
                             out_row.at[pl.ds(b, GE)], psem)
            return npat + 1

        npat = lax.cond(g != prevg, emit, lambda n: n, npat)
        return g, npat

    _, npat = lax.fori_loop(0, nlive, patch, (-1, 0))
    return npat


def _sc_body(x_hbm, out_hbm, xbuf0, xbuf1, cbuf, gmax, lcid, zrow,
             lsem0, lsem1, zsem, psem0, psem1):
    wid = lax.axis_index("s") * NC + lax.axis_index("c")
    base = wid * RPW
    xbuf = (xbuf0, xbuf1)
    lsem = (lsem0, lsem1)
    psem = (psem0, psem1)

    def z_init(g, _):
        b = g * GE
        for k in range(G):
            zrow[pl.ds(b + k * L, L)] = jnp.zeros((L,), jnp.float32)
        return 0
    lax.fori_loop(0, NGRP, z_init, 0)

    pltpu.async_copy(x_hbm.at[base], xbuf[0], lsem[0])

    def drain_patches(xb_other, out_row_other, sem, npat):
        def one(_, carry):
            pltpu.make_async_copy(xb_other.at[pl.ds(0, GE)],
                                  out_row_other.at[pl.ds(0, GE)],
                                  sem).wait()
            return carry
        lax.fori_loop(0, npat, one, 0)

    def outer(g, npats):
        npat0, npat1 = npats
        for b in (0, 1):
            r = g * 2 + b
            row = base + r
            xb = xbuf[b]
            nb = 1 - b
            npat_other = npat1 if b == 0 else npat0

            pltpu.async_copy(zrow, out_hbm.at[row], zsem)

            pltpu.make_async_copy(x_hbm.at[row], xb, lsem[b]).wait()

            rowmax, nlive = _phase1(xb, gmax, lcid)

            drain_patches(xbuf[nb], out_hbm.at[row - 1], psem[nb],
                          npat_other)
            if b == 0:
                pltpu.async_copy(x_hbm.at[row + 1], xbuf[nb], lsem[nb])
            else:
                def _prefetch():
                    pltpu.async_copy(x_hbm.at[row + 1], xbuf[nb], lsem[nb])
                    return None
                pl.when(g < (RPW // 2) - 1)(_prefetch)

            tau16 = _phase2(xb, cbuf, lcid, nlive, rowmax)

            pltpu.make_async_copy(zrow, out_hbm.at[row], zsem).wait()
            npat_new = _emit_patches(xb, lcid, nlive, tau16,
                                     out_hbm.at[row], psem[b])
            if b == 0:
                npat0 = npat_new
            else:
                npat1 = npat_new
        return npat0, npat1

    _, npat1 = lax.fori_loop(0, RPW // 2, outer, (0, 0))
    drain_patches(xbuf[1], out_hbm.at[base + RPW - 1], psem[1], npat1)


@jax.jit
def kernel(x):
    x2 = x.reshape(ROWS, N)
    mesh = plsc.VectorSubcoreMesh(
        core_axis_name="c", subcore_axis_name="s",
        num_cores=NC, num_subcores=NS)
    out = pl.kernel(
        _sc_body,
        out_type=jax.ShapeDtypeStruct((ROWS, N), jnp.float32),
        mesh=mesh,
        scratch_types=[
            pltpu.VMEM((N,), jnp.float32),
            pltpu.VMEM((N,), jnp.float32),
            pltpu.VMEM((C + L,), jnp.float32),
            pltpu.VMEM((NV,), jnp.float32),
            pltpu.VMEM((NV + L,), jnp.int32),
            pltpu.VMEM((N,), jnp.float32),
            pltpu.SemaphoreType.DMA,
            pltpu.SemaphoreType.DMA,
            pltpu.SemaphoreType.DMA,
            pltpu.SemaphoreType.DMA,
            pltpu.SemaphoreType.DMA,
        ],
        compiler_params=pltpu.CompilerParams(needs_layout_passes=False),
    )(x2)
    return out.reshape(x.shape)


# speedup vs baseline: 1.0068x; 1.0068x over previous
import jax
import jax.numpy as jnp
from jax import lax
from jax.experimental import pallas as pl
from jax.experimental.pallas import tpu as pltpu
from jax.experimental.pallas import tpu_sc as plsc

NC = 2
NS = 16
L = 16
NW = NC * NS

N = 32768
ROWS = 1024
RPW = ROWS // NW
NV = N // L

G = 16
NGRP = NV // G
GE = G * L

C = 4096

BISECT = 14
REFINE = 3

_NEG = -3.0e38


def _tau_from(buf, nv, rowmax):
    def relu_sum(t):
        def body(j, acc):
            v = buf[pl.ds(j * L, L)]
            return acc + jnp.maximum(v - t, 0.0)
        acc = lax.fori_loop(0, nv, body, jnp.zeros((L,), jnp.float32))
        return jnp.sum(acc)

    lo = rowmax - 1.0
    hi = rowmax

    def bis(_, lohi):
        lo, hi = lohi
        mid = 0.5 * (lo + hi)
        big = relu_sum(mid) >= 1.0
        return (jnp.where(big, mid, lo), jnp.where(big, hi, mid))

    lo, hi = lax.fori_loop(0, BISECT, bis, (lo, hi))

    def refine(_, t16):
        def body(j, carry):
            s16, k16 = carry
            v = buf[pl.ds(j * L, L)]
            m = v > t16
            return (s16 + jnp.where(m, v, 0.0),
                    k16 + jnp.where(m, 1.0, 0.0))
        s16, k16 = lax.fori_loop(
            0, nv, body,
            (jnp.zeros((L,), jnp.float32), jnp.zeros((L,), jnp.float32)))
        num = jnp.broadcast_to(jnp.sum(s16) - 1.0, (L,))
        den = jnp.broadcast_to(jnp.sum(k16), (L,))
        return jnp.maximum(t16, num / den)

    return lax.fori_loop(0, REFINE, refine, jnp.broadcast_to(lo, (L,)))


def _phase1(xb, gmax, lcid):
    neg = jnp.full((L,), _NEG, jnp.float32)

    @plsc.parallel_loop(0, NGRP, unroll=2, carry=neg)
    def grp_max(g, acc):
        b = g * GE
        v = [xb[pl.ds(b + k * L, L)] for k in range(G)]
        m = [jnp.maximum(v[2 * i], v[2 * i + 1]) for i in range(8)]
        m = [jnp.maximum(m[2 * i], m[2 * i + 1]) for i in range(4)]
        m = [jnp.maximum(m[2 * i], m[2 * i + 1]) for i in range(2)]
        cm = jnp.maximum(m[0], m[1])
        gmax[pl.ds(g * L, L)] = cm
        return jnp.maximum(acc, cm)

    rowmax = jnp.max(grp_max)
    thr16 = jnp.broadcast_to(rowmax - 1.0, (L,))

    def live_cols(gg, off):
        for k4 in range(4):
            g = gg * 4 + k4
            m = gmax[pl.ds(g * L, L)] > thr16
            ids = lax.iota(jnp.int32, L) + g * L
            plsc.store_compressed(lcid.at[pl.ds(off, L)], ids, mask=m)
            off = off + plsc.all_reduce_population_count(m)[0]
        return off

    nlive = lax.fori_loop(0, NGRP // 4, live_cols, 0)
    lcid[pl.ds(nlive, L)] = jnp.zeros((L,), jnp.int32)
    return rowmax, nlive


def _phase2(xb, cbuf, lcid, nlive, rowmax):
    thr16 = jnp.broadcast_to(rowmax - 1.0, (L,))
    nl16 = (nlive + (L - 1)) // L

    def gather_cols(i, carry):
        off, true_cnt = carry
        w = lcid[pl.ds(i * L, L)]
        lane_ok = (lax.iota(jnp.int32, L) + i * L) < nlive
        base = jnp.right_shift(w, 4) * GE + jnp.bitwise_and(w, 15)
        for k in range(G):
            val = plsc.load_gather(xb, [base + k * L])
            mk = jnp.logical_and(val > thr16, lane_ok)
            fits = jnp.broadcast_to(off <= C - L, (L,))
            ms = jnp.logical_and(mk, fits)
            plsc.store_compressed(cbuf.at[pl.ds(off, L)], val, mask=ms)
            off = off + plsc.all_reduce_population_count(ms)[0]
            true_cnt = true_cnt + plsc.all_reduce_population_count(mk)[0]
        return off, true_cnt

    cnt, true_cnt = lax.fori_loop(0, nl16, gather_cols, (0, 0))
    cbuf[pl.ds(cnt, L)] = jnp.full((L,), _NEG, jnp.float32)

    nvc = (cnt + (L - 1)) // L
    return lax.cond(
        true_cnt == cnt,
        lambda: _tau_from(cbuf, nvc, rowmax),
        lambda: _tau_from(xb, NV, rowmax))


def _emit_patches(xb, lcid, nlive, tau16, out_row, psem):
    def patch(i, carry):
        prevg, npat = carry
        g = jnp.right_shift(lcid[pl.ds(i, L)][0], 4)

        def emit(npat):
            b = g * GE
            for k in range(G):
                v = xb[pl.ds(b + k * L, L)]
                xb[pl.ds(b + k * L, L)] = jnp.maximum(v - tau16, 0.0)
            pltpu.async_copy(xb.at[pl.ds(b, GE)],
                             out_row.at[pl.ds(b, GE)], psem)
            return npat + 1

        npat = lax.cond(g != prevg, emit, lambda n: n, npat)
        return g, npat

    _, npat = lax.fori_loop(0, nlive, patch, (-1, 0))
    return npat


def _sc_body(x_hbm, out_hbm, xbuf0, xbuf1, cbuf, gmax, lcid, zrow,
             lsem0, lsem1, zsem, psem0, psem1):
    wid = lax.axis_index("s") * NC + lax.axis_index("c")
    base = wid * RPW
    xbuf = (xbuf0, xbuf1)
    lsem = (lsem0, lsem1)
    psem = (psem0, psem1)

    def z_init(g, _):
        b = g * GE
        for k in range(G):
            zrow[pl.ds(b + k * L, L)] = jnp.zeros((L,), jnp.float32)
        return 0
    lax.fori_loop(0, NGRP, z_init, 0)

    pltpu.async_copy(x_hbm.at[base], xbuf[0], lsem[0])

    def drain_patches(xb_other, out_row_other, sem, npat):
        def one(_, carry):
            pltpu.make_async_copy(xb_other.at[pl.ds(0, GE)],
                                  out_row_other.at[pl.ds(0, GE)],
                                  sem).wait()
            return carry
        lax.fori_loop(0, npat, one, 0)

    def outer(g, npats):
        npat0, npat1 = npats
        for b in (0, 1):
            r = g * 2 + b
            row = base + r
            xb = xbuf[b]
            nb = 1 - b
            npat_other = npat1 if b == 0 else npat0

            pltpu.async_copy(zrow, out_hbm.at[row], zsem)

            pltpu.make_async_copy(x_hbm.at[row], xb, lsem[b]).wait()

            rowmax, nlive = _phase1(xb, gmax, lcid)

            drain_patches(xbuf[nb], out_hbm.at[row - 1], psem[nb],
                          npat_other)
            if b == 0:
                pltpu.async_copy(x_hbm.at[row + 1], xbuf[nb], lsem[nb])
            else:
                def _prefetch():
                    pltpu.async_copy(x_hbm.at[row + 1], xbuf[nb], lsem[nb])
                    return None
                pl.when(g < (RPW // 2) - 1)(_prefetch)

            tau16 = _phase2(xb, cbuf, lcid, nlive, rowmax)

            pltpu.make_async_copy(zrow, out_hbm.at[row], zsem).wait()
            npat_new = _emit_patches(xb, lcid, nlive, tau16,
                                     out_hbm.at[row], psem[b])
            if b == 0:
                npat0 = npat_new
            else:
                npat1 = npat_new
        return npat0, npat1

    _, npat1 = lax.fori_loop(0, RPW // 2, outer, (0, 0))
    drain_patches(xbuf[1], out_hbm.at[base + RPW - 1], psem[1], npat1)


@jax.jit
def kernel(x):
    x2 = x.reshape(ROWS, N)
    mesh = plsc.VectorSubcoreMesh(
        core_axis_name="c", subcore_axis_name="s",
        num_cores=NC, num_subcores=NS)
    out = pl.kernel(
        _sc_body,
        out_type=jax.ShapeDtypeStruct((ROWS, N), jnp.float32),
        mesh=mesh,
        scratch_types=[
            pltpu.VMEM((N,), jnp.float32),
            pltpu.VMEM((N,), jnp.float32),
            pltpu.VMEM((C + L,), jnp.float32),
            pltpu.VMEM((NV,), jnp.float32),
            pltpu.VMEM((NV + L,), jnp.int32),
            pltpu.VMEM((N,), jnp.float32),
            pltpu.SemaphoreType.DMA,
            pltpu.SemaphoreType.DMA,
            pltpu.SemaphoreType.DMA,
            pltpu.SemaphoreType.DMA,
            pltpu.SemaphoreType.DMA,
        ],
        compiler_params=pltpu.CompilerParams(needs_layout_passes=False),
    )(x2)
    return out.reshape(x.shape)
